# baseline (speedup 1.0000x reference)
import functools

import jax
import jax.numpy as jnp
from jax import lax
from jax.experimental import pallas as pl
from jax.experimental.pallas import tpu as pltpu
from jax.experimental.pallas import tpu_sc as plsc

N = 10000
E = 320000
IN_CH = 128
HID = 64
OUT_CH = 5

NC = 2
NS = 16
NW = NC * NS
EPW = E // NW
CHUNK = 100
NCHUNK = EPW // CHUNK
N_PAD = 10240
RPT = N_PAD // NS


def _gelu_exact(v):
    return 0.5 * v * (1.0 + lax.erf(v * 0.7071067811865476))


def _make_edge_scatter(width):
    mesh = plsc.VectorSubcoreMesh(core_axis_name="c", subcore_axis_name="s")

    @functools.partial(
        pl.kernel,
        mesh=mesh,
        compiler_params=pltpu.CompilerParams(use_tc_tiling_on_sc=False),
        out_type=jax.ShapeDtypeStruct((NC, N_PAD, width), jnp.float32),
        scratch_types=[
            pltpu.VMEM((NCHUNK, CHUNK), jnp.int32),
            pltpu.VMEM((NCHUNK, CHUNK), jnp.int32),
            pltpu.VMEM((CHUNK, width), jnp.float32),
            pltpu.VMEM((CHUNK, width), jnp.float32),
            pltpu.VMEM_SHARED((N_PAD, width), jnp.float32),
            pltpu.SemaphoreType.DMA,
            pltpu.SemaphoreType.DMA,
        ],
    )
    def k(vals_hbm, src_hbm, dst_hbm, out_hbm, sidx, didx, rows0, rows1,
          acc, sem0, sem1):
        cid = lax.axis_index("c")
        sid = lax.axis_index("s")
        wid = sid * NC + cid

        ibase = wid * NCHUNK
        pltpu.async_copy(src_hbm.at[pl.ds(ibase, NCHUNK)], sidx, sem0)
        pltpu.async_copy(dst_hbm.at[pl.ds(ibase, NCHUNK)], didx, sem1)

        zvec = jnp.zeros((16,), jnp.float32)

        def zero_body(i, carry):
            for j in range(width // 16):
                rows0[i, pl.ds(j * 16, 16)] = zvec
            return carry

        lax.fori_loop(0, CHUNK, zero_body, 0)
        full, tail = divmod(RPT, CHUNK)
        for j in range(full):
            pltpu.sync_copy(rows0, acc.at[pl.ds(sid * RPT + j * CHUNK, CHUNK)])
        if tail:
            pltpu.sync_copy(rows0.at[pl.ds(0, tail)],
                            acc.at[pl.ds(sid * RPT + full * CHUNK, tail)])

        pltpu.make_async_copy(src_hbm.at[pl.ds(ibase, NCHUNK)], sidx, sem0).wait()
        pltpu.make_async_copy(dst_hbm.at[pl.ds(ibase, NCHUNK)], didx, sem1).wait()
        plsc.subcore_barrier()

        pltpu.async_copy(vals_hbm.at[sidx.at[0]], rows0, sem0)

        def edge_body(t, carry):
            c0 = 2 * t
            c1 = 2 * t + 1
            c2 = jnp.where(c1 + 1 >= NCHUNK, 0, c1 + 1)
            pltpu.make_async_copy(vals_hbm.at[sidx.at[c0]], rows0, sem0).wait()
            pltpu.async_copy(vals_hbm.at[sidx.at[c1]], rows1, sem1)
            pltpu.sync_copy(rows0, acc.at[didx.at[c0]], add=True)
            pltpu.make_async_copy(vals_hbm.at[sidx.at[c1]], rows1, sem1).wait()
            pltpu.async_copy(vals_hbm.at[sidx.at[c2]], rows0, sem0)
            pltpu.sync_copy(rows1, acc.at[didx.at[c1]], add=True)
            return carry

        lax.fori_loop(0, NCHUNK // 2, edge_body, 0)
        pltpu.make_async_copy(vals_hbm.at[sidx.at[0]], rows0, sem0).wait()
        plsc.subcore_barrier()
        pltpu.sync_copy(
            acc.at[pl.ds(sid * RPT, RPT)],
            out_hbm.at[cid, pl.ds(sid * RPT, RPT)],
        )

    return k


_scatter_in = _make_edge_scatter(IN_CH)
_scatter_hid = _make_edge_scatter(HID)


_ROWS_BLK = 1000
_GRID = N // _ROWS_BLK


def _dense_body(v_ref, w_ref, o_ref):
    o_ref[...] = jnp.dot(v_ref[...], w_ref[...],
                         preferred_element_type=jnp.float32)


def _tc1_body(a_ref, p0_ref, p1_ref, wn_ref, h_ref):
    agg = p0_ref[0] + p1_ref[0]
    pre = a_ref[...] + jnp.dot(agg, wn_ref[...],
                               preferred_element_type=jnp.float32)
    h_ref[...] = _gelu_exact(_gelu_exact(pre))


def _tc2_body(b_ref, q0_ref, q1_ref, wn_ref, o_ref):
    agg = q0_ref[0] + q1_ref[0]
    pre = b_ref[...] + jnp.dot(agg, wn_ref[...],
                               preferred_element_type=jnp.float32)
    o_ref[...] = jnp.tanh(_gelu_exact(pre))


def kernel(x, edge_index, Ws0, Wn0, Ws1, Wn1):
    src = edge_index[0].astype(jnp.int32).reshape(E // CHUNK, CHUNK)
    dst = edge_index[1].astype(jnp.int32).reshape(E // CHUNK, CHUNK)

    row_spec = lambda w: pl.BlockSpec((_ROWS_BLK, w), lambda i: (i, 0))
    full_spec = lambda r, c: pl.BlockSpec((r, c), lambda i: (0, 0))
    part_spec = lambda w, c: pl.BlockSpec(
        (1, _ROWS_BLK, w), lambda i, c=c: (c, i, 0))

    dense = lambda v, w, ci, co: pl.pallas_call(
        _dense_body,
        grid=(_GRID,),
        in_specs=[row_spec(ci), full_spec(ci, co)],
        out_specs=row_spec(co),
        out_shape=jax.ShapeDtypeStruct((N, co), jnp.float32),
    )(v, w)

    a0 = dense(x, Ws0.T, IN_CH, HID)
    parts0 = _scatter_in(x, src, dst)

    h1 = pl.pallas_call(
        _tc1_body,
        grid=(_GRID,),
        in_specs=[row_spec(HID), part_spec(IN_CH, 0), part_spec(IN_CH, 1),
                  full_spec(IN_CH, HID)],
        out_specs=row_spec(HID),
        out_shape=jax.ShapeDtypeStruct((N, HID), jnp.float32),
    )(a0, parts0, parts0, Wn0.T)

    b0 = dense(h1, Ws1.T, HID, OUT_CH)
    parts1 = _scatter_hid(h1, src, dst)

    out = pl.pallas_call(
        _tc2_body,
        grid=(_GRID,),
        in_specs=[row_spec(OUT_CH), part_spec(HID, 0), part_spec(HID, 1),
                  full_spec(HID, OUT_CH)],
        out_specs=row_spec(OUT_CH),
        out_shape=jax.ShapeDtypeStruct((N, OUT_CH), jnp.float32),
    )(b0, parts1, parts1, Wn1.T)

    return out


# speedup vs baseline: 1.1911x; 1.1911x over previous
import functools

import jax
import jax.numpy as jnp
from jax import lax
from jax.experimental import pallas as pl
from jax.experimental.pallas import tpu as pltpu
from jax.experimental.pallas import tpu_sc as plsc

N = 10000
E = 320000
IN_CH = 128
HID = 64
OUT_CH = 5

NC = 2
NS = 16
NW = NC * NS
EPW = E // NW
CHUNK = 100
NCHUNK = EPW // CHUNK
N_PAD = 10240
RPT = N_PAD // NS


def _gelu_exact(v):
    return 0.5 * v * (1.0 + lax.erf(v * 0.7071067811865476))


def _make_edge_scatter(width, nbuf):
    assert NCHUNK % nbuf == 0
    pd = nbuf - 1
    mesh = plsc.VectorSubcoreMesh(core_axis_name="c", subcore_axis_name="s")

    @functools.partial(
        pl.kernel,
        mesh=mesh,
        compiler_params=pltpu.CompilerParams(use_tc_tiling_on_sc=False),
        out_type=jax.ShapeDtypeStruct((NC, N_PAD, width), jnp.float32),
        scratch_types=(
            [pltpu.VMEM((NCHUNK, CHUNK), jnp.int32),
             pltpu.VMEM((NCHUNK, CHUNK), jnp.int32)]
            + [pltpu.VMEM((CHUNK, width), jnp.float32)] * nbuf
            + [pltpu.VMEM_SHARED((N_PAD, width), jnp.float32)]
            + [pltpu.SemaphoreType.DMA] * nbuf
        ),
    )
    def k(vals_hbm, src_hbm, dst_hbm, out_hbm, sidx, didx, *rest):
        rows = rest[:nbuf]
        acc = rest[nbuf]
        sems = rest[nbuf + 1:]
        cid = lax.axis_index("c")
        sid = lax.axis_index("s")
        wid = sid * NC + cid

        ibase = wid * NCHUNK
        pltpu.async_copy(src_hbm.at[pl.ds(ibase, NCHUNK)], sidx, sems[0])
        pltpu.async_copy(dst_hbm.at[pl.ds(ibase, NCHUNK)], didx, sems[1])

        zvec = jnp.zeros((16,), jnp.float32)

        def zero_body(i, carry):
            for j in range(width // 16):
                rows[0][i, pl.ds(j * 16, 16)] = zvec
            return carry

        lax.fori_loop(0, CHUNK, zero_body, 0)
        full, tail = divmod(RPT, CHUNK)
        for j in range(full):
            pltpu.sync_copy(rows[0],
                            acc.at[pl.ds(sid * RPT + j * CHUNK, CHUNK)])
        if tail:
            pltpu.sync_copy(rows[0].at[pl.ds(0, tail)],
                            acc.at[pl.ds(sid * RPT + full * CHUNK, tail)])

        pltpu.make_async_copy(src_hbm.at[pl.ds(ibase, NCHUNK)], sidx,
                              sems[0]).wait()
        pltpu.make_async_copy(dst_hbm.at[pl.ds(ibase, NCHUNK)], didx,
                              sems[1]).wait()
        plsc.subcore_barrier()

        for b in range(pd):
            pltpu.async_copy(vals_hbm.at[sidx.at[b]], rows[b], sems[b])

        def edge_body(t, carry):
            base = nbuf * t
            for j in range(nbuf):
                c = base + j
                nxt = c + pd
                pn = jnp.where(nxt >= NCHUNK, nxt - NCHUNK, nxt)
                bj = (j + pd) % nbuf
                pltpu.make_async_copy(vals_hbm.at[sidx.at[c]], rows[j],
                                      sems[j]).wait()
                pltpu.async_copy(vals_hbm.at[sidx.at[pn]], rows[bj], sems[bj])
                pltpu.sync_copy(rows[j], acc.at[didx.at[c]], add=True)
            return carry

        lax.fori_loop(0, NCHUNK // nbuf, edge_body, 0)
        for b in range(pd):
            pltpu.make_async_copy(vals_hbm.at[sidx.at[b]], rows[b],
                                  sems[b]).wait()
        plsc.subcore_barrier()
        pltpu.sync_copy(
            acc.at[pl.ds(sid * RPT, RPT)],
            out_hbm.at[cid, pl.ds(sid * RPT, RPT)],
        )

    return k


_scatter_in = _make_edge_scatter(IN_CH, 2)
_scatter_hid = _make_edge_scatter(HID, 4)


_ROWS_BLK = 1000
_GRID = N // _ROWS_BLK


def _tc1_body(x_ref, p0_ref, p1_ref, ws_ref, wn_ref, h_ref):
    agg = p0_ref[0] + p1_ref[0]
    pre = (jnp.dot(x_ref[...], ws_ref[...], preferred_element_type=jnp.float32)
           + jnp.dot(agg, wn_ref[...], preferred_element_type=jnp.float32))
    h_ref[...] = _gelu_exact(_gelu_exact(pre))


def _tc2_body(h_ref, q0_ref, q1_ref, ws_ref, wn_ref, o_ref):
    agg = q0_ref[0] + q1_ref[0]
    pre = (jnp.dot(h_ref[...], ws_ref[...], preferred_element_type=jnp.float32)
           + jnp.dot(agg, wn_ref[...], preferred_element_type=jnp.float32))
    o_ref[...] = jnp.tanh(_gelu_exact(pre))


def kernel(x, edge_index, Ws0, Wn0, Ws1, Wn1):
    src = edge_index[0].astype(jnp.int32).reshape(E // CHUNK, CHUNK)
    dst = edge_index[1].astype(jnp.int32).reshape(E // CHUNK, CHUNK)

    row_spec = lambda w: pl.BlockSpec((_ROWS_BLK, w), lambda i: (i, 0))
    full_spec = lambda r, c: pl.BlockSpec((r, c), lambda i: (0, 0))
    part_spec = lambda w, c: pl.BlockSpec(
        (1, _ROWS_BLK, w), lambda i, c=c: (c, i, 0))

    parts0 = _scatter_in(x, src, dst)

    h1 = pl.pallas_call(
        _tc1_body,
        grid=(_GRID,),
        in_specs=[row_spec(IN_CH), part_spec(IN_CH, 0), part_spec(IN_CH, 1),
                  full_spec(IN_CH, HID), full_spec(IN_CH, HID)],
        out_specs=row_spec(HID),
        out_shape=jax.ShapeDtypeStruct((N, HID), jnp.float32),
    )(x, parts0, parts0, Ws0.T, Wn0.T)

    parts1 = _scatter_hid(h1, src, dst)

    out = pl.pallas_call(
        _tc2_body,
        grid=(_GRID,),
        in_specs=[row_spec(HID), part_spec(HID, 0), part_spec(HID, 1),
                  full_spec(HID, OUT_CH), full_spec(HID, OUT_CH)],
        out_specs=row_spec(OUT_CH),
        out_shape=jax.ShapeDtypeStruct((N, OUT_CH), jnp.float32),
    )(h1, parts1, parts1, Ws1.T, Wn1.T)

    return out


# speedup vs baseline: 1.3361x; 1.1217x over previous
import functools

import jax
import jax.numpy as jnp
from jax import lax
from jax.experimental import pallas as pl
from jax.experimental.pallas import tpu as pltpu
from jax.experimental.pallas import tpu_sc as plsc

N = 10000
E = 320000
IN_CH = 128
HID = 64
OUT_CH = 5

NC = 2
NS = 16
NW = NC * NS
EPW = E // NW
N_PAD = 10240
RPT = N_PAD // NS
CHUNK1 = 50
CHUNK2 = 100


def _gelu_exact(v):
    return 0.5 * v * (1.0 + lax.erf(v * 0.7071067811865476))


def _make_edge_scatter(width, nbuf, chunk):
    NCHUNK = EPW // chunk
    CHUNK = chunk
    assert EPW % chunk == 0 and NCHUNK % nbuf == 0
    pd = nbuf - 1
    mesh = plsc.VectorSubcoreMesh(core_axis_name="c", subcore_axis_name="s")

    @functools.partial(
        pl.kernel,
        mesh=mesh,
        compiler_params=pltpu.CompilerParams(use_tc_tiling_on_sc=False),
        out_type=jax.ShapeDtypeStruct((NC, N_PAD, width), jnp.float32),
        scratch_types=(
            [pltpu.VMEM((NCHUNK, CHUNK), jnp.int32),
             pltpu.VMEM((NCHUNK, CHUNK), jnp.int32)]
            + [pltpu.VMEM((CHUNK, width), jnp.float32)] * nbuf
            + [pltpu.VMEM_SHARED((N_PAD, width), jnp.float32)]
            + [pltpu.SemaphoreType.DMA] * nbuf
        ),
    )
    def k(vals_hbm, src_hbm, dst_hbm, out_hbm, sidx, didx, *rest):
        rows = rest[:nbuf]
        acc = rest[nbuf]
        sems = rest[nbuf + 1:]
        cid = lax.axis_index("c")
        sid = lax.axis_index("s")
        wid = sid * NC + cid

        ibase = wid * NCHUNK
        pltpu.async_copy(src_hbm.at[pl.ds(ibase, NCHUNK)], sidx, sems[0])
        pltpu.async_copy(dst_hbm.at[pl.ds(ibase, NCHUNK)], didx, sems[1])

        zvec = jnp.zeros((16,), jnp.float32)

        def zero_body(i, carry):
            for j in range(width // 16):
                rows[0][i, pl.ds(j * 16, 16)] = zvec
            return carry

        lax.fori_loop(0, CHUNK, zero_body, 0)
        full, tail = divmod(RPT, CHUNK)
        for j in range(full):
            pltpu.sync_copy(rows[0],
                            acc.at[pl.ds(sid * RPT + j * CHUNK, CHUNK)])
        if tail:
            pltpu.sync_copy(rows[0].at[pl.ds(0, tail)],
                            acc.at[pl.ds(sid * RPT + full * CHUNK, tail)])

        pltpu.make_async_copy(src_hbm.at[pl.ds(ibase, NCHUNK)], sidx,
                              sems[0]).wait()
        pltpu.make_async_copy(dst_hbm.at[pl.ds(ibase, NCHUNK)], didx,
                              sems[1]).wait()
        plsc.subcore_barrier()

        for b in range(pd):
            pltpu.async_copy(vals_hbm.at[sidx.at[b]], rows[b], sems[b])

        def edge_body(t, carry):
            base = nbuf * t
            for j in range(nbuf):
                c = base + j
                nxt = c + pd
                pn = jnp.where(nxt >= NCHUNK, nxt - NCHUNK, nxt)
                bj = (j + pd) % nbuf
                pltpu.make_async_copy(vals_hbm.at[sidx.at[c]], rows[j],
                                      sems[j]).wait()
                pltpu.async_copy(vals_hbm.at[sidx.at[pn]], rows[bj], sems[bj])
                pltpu.sync_copy(rows[j], acc.at[didx.at[c]], add=True)
            return carry

        lax.fori_loop(0, NCHUNK // nbuf, edge_body, 0)
        for b in range(pd):
            pltpu.make_async_copy(vals_hbm.at[sidx.at[b]], rows[b],
                                  sems[b]).wait()
        plsc.subcore_barrier()
        pltpu.sync_copy(
            acc.at[pl.ds(sid * RPT, RPT)],
            out_hbm.at[cid, pl.ds(sid * RPT, RPT)],
        )

    return k


_scatter_in = _make_edge_scatter(IN_CH, 4, CHUNK1)
_scatter_hid = _make_edge_scatter(HID, 4, CHUNK2)


_ROWS_BLK = 1000
_GRID = N // _ROWS_BLK


def _tc1_body(x_ref, p0_ref, p1_ref, ws_ref, wn_ref, h_ref):
    agg = p0_ref[0] + p1_ref[0]
    pre = (jnp.dot(x_ref[...], ws_ref[...], preferred_element_type=jnp.float32)
           + jnp.dot(agg, wn_ref[...], preferred_element_type=jnp.float32))
    h_ref[...] = _gelu_exact(_gelu_exact(pre))


def _tc2_body(h_ref, q0_ref, q1_ref, ws_ref, wn_ref, o_ref):
    agg = q0_ref[0] + q1_ref[0]
    pre = (jnp.dot(h_ref[...], ws_ref[...], preferred_element_type=jnp.float32)
           + jnp.dot(agg, wn_ref[...], preferred_element_type=jnp.float32))
    o_ref[...] = jnp.tanh(_gelu_exact(pre))


def kernel(x, edge_index, Ws0, Wn0, Ws1, Wn1):
    src = edge_index[0].astype(jnp.int32)
    dst = edge_index[1].astype(jnp.int32)
    src1 = src.reshape(E // CHUNK1, CHUNK1)
    dst1 = dst.reshape(E // CHUNK1, CHUNK1)
    src2 = src.reshape(E // CHUNK2, CHUNK2)
    dst2 = dst.reshape(E // CHUNK2, CHUNK2)

    row_spec = lambda w: pl.BlockSpec((_ROWS_BLK, w), lambda i: (i, 0))
    full_spec = lambda r, c: pl.BlockSpec((r, c), lambda i: (0, 0))
    part_spec = lambda w, c: pl.BlockSpec(
        (1, _ROWS_BLK, w), lambda i, c=c: (c, i, 0))

    parts0 = _scatter_in(x, src1, dst1)

    h1 = pl.pallas_call(
        _tc1_body,
        grid=(_GRID,),
        in_specs=[row_spec(IN_CH), part_spec(IN_CH, 0), part_spec(IN_CH, 1),
                  full_spec(IN_CH, HID), full_spec(IN_CH, HID)],
        out_specs=row_spec(HID),
        out_shape=jax.ShapeDtypeStruct((N, HID), jnp.float32),
    )(x, parts0, parts0, Ws0.T, Wn0.T)

    parts1 = _scatter_hid(h1, src2, dst2)

    out = pl.pallas_call(
        _tc2_body,
        grid=(_GRID,),
        in_specs=[row_spec(HID), part_spec(HID, 0), part_spec(HID, 1),
                  full_spec(HID, OUT_CH), full_spec(HID, OUT_CH)],
        out_specs=row_spec(OUT_CH),
        out_shape=jax.ShapeDtypeStruct((N, OUT_CH), jnp.float32),
    )(h1, parts1, parts1, Ws1.T, Wn1.T)

    return out


# speedup vs baseline: 1.4901x; 1.1153x over previous
import functools

import jax
import jax.numpy as jnp
from jax import lax
from jax.experimental import pallas as pl
from jax.experimental.pallas import tpu as pltpu
from jax.experimental.pallas import tpu_sc as plsc

N = 10000
E = 320000
IN_CH = 128
HID = 64
OUT_CH = 5

NC = 2
NS = 16
NW = NC * NS
EPW = E // NW
N_PAD = 10240
RPT = N_PAD // NS
CHUNK1 = 40
CHUNK2 = 50


def _gelu_exact(v):
    return 0.5 * v * (1.0 + lax.erf(v * 0.7071067811865476))


def _make_edge_scatter(width, nbuf, chunk):
    NCHUNK = EPW // chunk
    CHUNK = chunk
    assert EPW % chunk == 0 and NCHUNK % nbuf == 0
    pd = nbuf - 1
    mesh = plsc.VectorSubcoreMesh(core_axis_name="c", subcore_axis_name="s")

    @functools.partial(
        pl.kernel,
        mesh=mesh,
        compiler_params=pltpu.CompilerParams(use_tc_tiling_on_sc=False),
        out_type=jax.ShapeDtypeStruct((NC, N_PAD, width), jnp.float32),
        scratch_types=(
            [pltpu.VMEM((NCHUNK, CHUNK), jnp.int32),
             pltpu.VMEM((NCHUNK, CHUNK), jnp.int32)]
            + [pltpu.VMEM((CHUNK, width), jnp.float32)] * nbuf
            + [pltpu.VMEM_SHARED((N_PAD, width), jnp.float32)]
            + [pltpu.SemaphoreType.DMA] * nbuf
        ),
    )
    def k(vals_hbm, src_hbm, dst_hbm, out_hbm, sidx, didx, *rest):
        rows = rest[:nbuf]
        acc = rest[nbuf]
        sems = rest[nbuf + 1:]
        cid = lax.axis_index("c")
        sid = lax.axis_index("s")
        wid = sid * NC + cid

        ibase = wid * NCHUNK
        pltpu.async_copy(src_hbm.at[pl.ds(ibase, NCHUNK)], sidx, sems[0])
        pltpu.async_copy(dst_hbm.at[pl.ds(ibase, NCHUNK)], didx, sems[1])

        zvec = jnp.zeros((16,), jnp.float32)

        def zero_body(i, carry):
            for j in range(width // 16):
                rows[0][i, pl.ds(j * 16, 16)] = zvec
            return carry

        lax.fori_loop(0, CHUNK, zero_body, 0)
        full, tail = divmod(RPT, CHUNK)
        for j in range(full):
            pltpu.sync_copy(rows[0],
                            acc.at[pl.ds(sid * RPT + j * CHUNK, CHUNK)])
        if tail:
            pltpu.sync_copy(rows[0].at[pl.ds(0, tail)],
                            acc.at[pl.ds(sid * RPT + full * CHUNK, tail)])

        pltpu.make_async_copy(src_hbm.at[pl.ds(ibase, NCHUNK)], sidx,
                              sems[0]).wait()
        pltpu.make_async_copy(dst_hbm.at[pl.ds(ibase, NCHUNK)], didx,
                              sems[1]).wait()
        plsc.subcore_barrier()

        for b in range(pd):
            pltpu.async_copy(vals_hbm.at[sidx.at[b]], rows[b], sems[b])

        def edge_body(t, carry):
            base = nbuf * t
            for j in range(nbuf):
                c = base + j
                nxt = c + pd
                pn = jnp.where(nxt >= NCHUNK, nxt - NCHUNK, nxt)
                bj = (j + pd) % nbuf
                pltpu.make_async_copy(vals_hbm.at[sidx.at[c]], rows[j],
                                      sems[j]).wait()
                pltpu.async_copy(vals_hbm.at[sidx.at[pn]], rows[bj], sems[bj])
                pltpu.sync_copy(rows[j], acc.at[didx.at[c]], add=True)
            return carry

        lax.fori_loop(0, NCHUNK // nbuf, edge_body, 0)
        for b in range(pd):
            pltpu.make_async_copy(vals_hbm.at[sidx.at[b]], rows[b],
                                  sems[b]).wait()
        plsc.subcore_barrier()
        pltpu.sync_copy(
            acc.at[pl.ds(sid * RPT, RPT)],
            out_hbm.at[cid, pl.ds(sid * RPT, RPT)],
        )

    return k


_scatter_in = _make_edge_scatter(IN_CH, 5, CHUNK1)
_scatter_hid = _make_edge_scatter(HID, 8, CHUNK2)


_ROWS_BLK = 1000
_GRID = N // _ROWS_BLK


def _tc1_body(x_ref, p0_ref, p1_ref, ws_ref, wn_ref, h_ref):
    agg = p0_ref[0] + p1_ref[0]
    pre = (jnp.dot(x_ref[...], ws_ref[...], preferred_element_type=jnp.float32)
           + jnp.dot(agg, wn_ref[...], preferred_element_type=jnp.float32))
    h_ref[...] = _gelu_exact(_gelu_exact(pre))


def _tc2_body(h_ref, q0_ref, q1_ref, ws_ref, wn_ref, o_ref):
    agg = q0_ref[0] + q1_ref[0]
    pre = (jnp.dot(h_ref[...], ws_ref[...], preferred_element_type=jnp.float32)
           + jnp.dot(agg, wn_ref[...], preferred_element_type=jnp.float32))
    o_ref[...] = jnp.tanh(_gelu_exact(pre))


def kernel(x, edge_index, Ws0, Wn0, Ws1, Wn1):
    src = edge_index[0].astype(jnp.int32)
    dst = edge_index[1].astype(jnp.int32)
    src1 = src.reshape(E // CHUNK1, CHUNK1)
    dst1 = dst.reshape(E // CHUNK1, CHUNK1)
    src2 = src.reshape(E // CHUNK2, CHUNK2)
    dst2 = dst.reshape(E // CHUNK2, CHUNK2)

    row_spec = lambda w: pl.BlockSpec((_ROWS_BLK, w), lambda i: (i, 0))
    full_spec = lambda r, c: pl.BlockSpec((r, c), lambda i: (0, 0))
    part_spec = lambda w, c: pl.BlockSpec(
        (1, _ROWS_BLK, w), lambda i, c=c: (c, i, 0))

    parts0 = _scatter_in(x, src1, dst1)

    h1 = pl.pallas_call(
        _tc1_body,
        grid=(_GRID,),
        in_specs=[row_spec(IN_CH), part_spec(IN_CH, 0), part_spec(IN_CH, 1),
                  full_spec(IN_CH, HID), full_spec(IN_CH, HID)],
        out_specs=row_spec(HID),
        out_shape=jax.ShapeDtypeStruct((N, HID), jnp.float32),
    )(x, parts0, parts0, Ws0.T, Wn0.T)

    parts1 = _scatter_hid(h1, src2, dst2)

    out = pl.pallas_call(
        _tc2_body,
        grid=(_GRID,),
        in_specs=[row_spec(HID), part_spec(HID, 0), part_spec(HID, 1),
                  full_spec(HID, OUT_CH), full_spec(HID, OUT_CH)],
        out_specs=row_spec(OUT_CH),
        out_shape=jax.ShapeDtypeStruct((N, OUT_CH), jnp.float32),
    )(h1, parts1, parts1, Ws1.T, Wn1.T)

    return out


# speedup vs baseline: 1.4918x; 1.0011x over previous
import functools

import jax
import jax.numpy as jnp
from jax import lax
from jax.experimental import pallas as pl
from jax.experimental.pallas import tpu as pltpu
from jax.experimental.pallas import tpu_sc as plsc

N = 10000
E = 320000
IN_CH = 128
HID = 64
OUT_CH = 5

NC = 2
NS = 16
NW = NC * NS
EPW = E // NW
N_PAD = 10240
RPT = N_PAD // NS
CHUNK1 = 40
CHUNK2 = 50


def _gelu_exact(v):
    return 0.5 * v * (1.0 + lax.erf(v * 0.7071067811865476))


def _make_edge_scatter(width, nbuf, chunk):
    NCHUNK = EPW // chunk
    CHUNK = chunk
    assert EPW % chunk == 0 and NCHUNK % nbuf == 0
    pd = nbuf - 1
    mesh = plsc.VectorSubcoreMesh(core_axis_name="c", subcore_axis_name="s")

    @functools.partial(
        pl.kernel,
        mesh=mesh,
        compiler_params=pltpu.CompilerParams(use_tc_tiling_on_sc=False),
        out_type=jax.ShapeDtypeStruct((NC, N_PAD, width), jnp.float32),
        scratch_types=(
            [pltpu.VMEM((NCHUNK, CHUNK), jnp.int32),
             pltpu.VMEM((NCHUNK, CHUNK), jnp.int32)]
            + [pltpu.VMEM((CHUNK, width), jnp.float32)] * nbuf
            + [pltpu.VMEM_SHARED((N_PAD, width), jnp.float32)]
            + [pltpu.SemaphoreType.DMA] * nbuf
        ),
    )
    def k(vals_hbm, src_hbm, dst_hbm, out_hbm, sidx, didx, *rest):
        rows = rest[:nbuf]
        acc = rest[nbuf]
        sems = rest[nbuf + 1:]
        cid = lax.axis_index("c")
        sid = lax.axis_index("s")
        wid = sid * NC + cid

        ibase = wid * NCHUNK
        pltpu.async_copy(src_hbm.at[pl.ds(ibase, NCHUNK)], sidx, sems[0])
        pltpu.async_copy(dst_hbm.at[pl.ds(ibase, NCHUNK)], didx, sems[1])

        zvec = jnp.zeros((16,), jnp.float32)

        def zero_body(i, carry):
            for j in range(width // 16):
                rows[0][i, pl.ds(j * 16, 16)] = zvec
            return carry

        lax.fori_loop(0, CHUNK, zero_body, 0)
        full, tail = divmod(RPT, CHUNK)
        for j in range(full):
            pltpu.sync_copy(rows[0],
                            acc.at[pl.ds(sid * RPT + j * CHUNK, CHUNK)])
        if tail:
            pltpu.sync_copy(rows[0].at[pl.ds(0, tail)],
                            acc.at[pl.ds(sid * RPT + full * CHUNK, tail)])

        pltpu.make_async_copy(src_hbm.at[pl.ds(ibase, NCHUNK)], sidx,
                              sems[0]).wait()
        pltpu.make_async_copy(dst_hbm.at[pl.ds(ibase, NCHUNK)], didx,
                              sems[1]).wait()
        plsc.subcore_barrier()

        for b in range(pd):
            pltpu.async_copy(vals_hbm.at[sidx.at[b]], rows[b], sems[b])

        def edge_body(t, carry):
            base = nbuf * t
            for j in range(nbuf):
                c = base + j
                nxt = c + pd
                pn = jnp.where(nxt >= NCHUNK, nxt - NCHUNK, nxt)
                bj = (j + pd) % nbuf
                pltpu.make_async_copy(vals_hbm.at[sidx.at[c]], rows[j],
                                      sems[j]).wait()
                pltpu.async_copy(vals_hbm.at[sidx.at[pn]], rows[bj], sems[bj])
                pltpu.sync_copy(rows[j], acc.at[didx.at[c]], add=True)
            return carry

        lax.fori_loop(0, NCHUNK // nbuf, edge_body, 0)
        for b in range(pd):
            pltpu.make_async_copy(vals_hbm.at[sidx.at[b]], rows[b],
                                  sems[b]).wait()
        plsc.subcore_barrier()
        pltpu.sync_copy(
            acc.at[pl.ds(sid * RPT, RPT)],
            out_hbm.at[cid, pl.ds(sid * RPT, RPT)],
        )

    return k


_scatter_in = _make_edge_scatter(IN_CH, 5, CHUNK1)
_scatter_hid = _make_edge_scatter(HID, 10, CHUNK2)


_ROWS_BLK = 1000
_GRID = N // _ROWS_BLK


def _tc1_body(x_ref, p0_ref, p1_ref, ws_ref, wn_ref, h_ref):
    agg = p0_ref[0] + p1_ref[0]
    pre = (jnp.dot(x_ref[...], ws_ref[...], preferred_element_type=jnp.float32)
           + jnp.dot(agg, wn_ref[...], preferred_element_type=jnp.float32))
    h_ref[...] = _gelu_exact(_gelu_exact(pre))


def _tc2_body(h_ref, q0_ref, q1_ref, ws_ref, wn_ref, o_ref):
    agg = q0_ref[0] + q1_ref[0]
    pre = (jnp.dot(h_ref[...], ws_ref[...], preferred_element_type=jnp.float32)
           + jnp.dot(agg, wn_ref[...], preferred_element_type=jnp.float32))
    o_ref[...] = jnp.tanh(_gelu_exact(pre))


def kernel(x, edge_index, Ws0, Wn0, Ws1, Wn1):
    src = edge_index[0].astype(jnp.int32)
    dst = edge_index[1].astype(jnp.int32)
    src1 = src.reshape(E // CHUNK1, CHUNK1)
    dst1 = dst.reshape(E // CHUNK1, CHUNK1)
    src2 = src.reshape(E // CHUNK2, CHUNK2)
    dst2 = dst.reshape(E // CHUNK2, CHUNK2)

    row_spec = lambda w: pl.BlockSpec((_ROWS_BLK, w), lambda i: (i, 0))
    full_spec = lambda r, c: pl.BlockSpec((r, c), lambda i: (0, 0))
    part_spec = lambda w, c: pl.BlockSpec(
        (1, _ROWS_BLK, w), lambda i, c=c: (c, i, 0))

    parts0 = _scatter_in(x, src1, dst1)

    h1 = pl.pallas_call(
        _tc1_body,
        grid=(_GRID,),
        in_specs=[row_spec(IN_CH), part_spec(IN_CH, 0), part_spec(IN_CH, 1),
                  full_spec(IN_CH, HID), full_spec(IN_CH, HID)],
        out_specs=row_spec(HID),
        out_shape=jax.ShapeDtypeStruct((N, HID), jnp.float32),
    )(x, parts0, parts0, Ws0.T, Wn0.T)

    parts1 = _scatter_hid(h1, src2, dst2)

    out = pl.pallas_call(
        _tc2_body,
        grid=(_GRID,),
        in_specs=[row_spec(HID), part_spec(HID, 0), part_spec(HID, 1),
                  full_spec(HID, OUT_CH), full_spec(HID, OUT_CH)],
        out_specs=row_spec(OUT_CH),
        out_shape=jax.ShapeDtypeStruct((N, OUT_CH), jnp.float32),
    )(h1, parts1, parts1, Ws1.T, Wn1.T)

    return out
